# baseline (speedup 1.0000x reference)
import jax
import jax.numpy as jnp
import numpy as np
from jax.experimental import pallas as pl
from jax.experimental.pallas import tpu as pltpu


def _dct_mat(n):
    i = np.arange(n, dtype=np.float64)
    k = np.arange(n, dtype=np.float64)[:, None]
    m = np.cos(np.pi * (2.0 * i[None, :] + 1.0) * k / (2.0 * n))
    s = np.where(k == 0, np.sqrt(1.0 / n), np.sqrt(2.0 / n))
    return m * s


_T = np.asarray(_dct_mat(32)[:28, :].T @ _dct_mat(64)[:28, :], dtype=np.float32)


_S = 8


def _body(t_ref, x_ref, o_ref):
    t = t_ref[...]
    x = x_ref[...]
    s = x.shape[0]
    nt = (((1,), (1,)), ((), ()))
    on = (((1,), (0,)), ((), ()))
    hi = jax.lax.Precision.DEFAULT
    a = jax.lax.dot_general(x.reshape(s * 64 * 64, 64), t, nt, precision=hi,
                            preferred_element_type=jnp.float32)
    a = jnp.swapaxes(a.reshape(s * 64, 64, 32).astype(jnp.bfloat16), 1, 2)
    b = jax.lax.dot_general(a.reshape(s * 64 * 32, 64), t.astype(jnp.bfloat16), nt, precision=hi,
                            preferred_element_type=jnp.float32)
    b = b.reshape(s, 64, 32, 32).astype(jnp.bfloat16)
    bc = (((1,), (2,)), ((0,), (0,)))
    tb = jnp.broadcast_to(t, (s, 32, 64)).astype(jnp.bfloat16)
    c = jax.lax.dot_general(b, tb, bc, precision=hi,
                            preferred_element_type=jnp.float32)
    o_ref[...] = c.astype(jnp.bfloat16)


def kernel(x):
    b, ch = x.shape[0], x.shape[1]
    n = b * ch
    xf = x.reshape(n, 64, 64, 64)
    out = pl.pallas_call(
        _body,
        grid=(n // _S,),
        in_specs=[
            pl.BlockSpec((32, 64), lambda i: (0, 0)),
            pl.BlockSpec((_S, 64, 64, 64), lambda i: (i, 0, 0, 0)),
        ],
        out_specs=pl.BlockSpec((_S, 32, 32, 32), lambda i: (i, 0, 0, 0)),
        out_shape=jax.ShapeDtypeStruct((n, 32, 32, 32), jnp.bfloat16),
        compiler_params=pltpu.CompilerParams(
            dimension_semantics=("parallel",),
        ),
    )(jnp.asarray(_T), xf)
    out = jnp.transpose(out, (0, 3, 2, 1)).astype(jnp.float32)
    return out.reshape(b, ch, 32, 32, 32)


# speedup vs baseline: 1.1031x; 1.1031x over previous
import jax
import jax.numpy as jnp
import numpy as np
from jax.experimental import pallas as pl
from jax.experimental.pallas import tpu as pltpu


def _dct_mat(n):
    i = np.arange(n, dtype=np.float64)
    k = np.arange(n, dtype=np.float64)[:, None]
    m = np.cos(np.pi * (2.0 * i[None, :] + 1.0) * k / (2.0 * n))
    s = np.where(k == 0, np.sqrt(1.0 / n), np.sqrt(2.0 / n))
    return m * s


_T = np.asarray(_dct_mat(32)[:28, :].T @ _dct_mat(64)[:28, :], dtype=np.float32)


_S = 8


def _half(x, t, tb_half):
    s = x.shape[0]
    nt = (((1,), (1,)), ((), ()))
    hi = jax.lax.Precision.DEFAULT
    a = jax.lax.dot_general(x.reshape(s * 32 * 64, 64), t, nt, precision=hi,
                            preferred_element_type=jnp.float32)
    a = jnp.swapaxes(a.reshape(s * 32, 64, 32).astype(jnp.bfloat16), 1, 2)
    b = jax.lax.dot_general(a.reshape(s * 32 * 32, 64),
                            t.astype(jnp.bfloat16), nt, precision=hi,
                            preferred_element_type=jnp.float32)
    b = b.reshape(s, 32, 32, 32).astype(jnp.bfloat16)
    bc = (((1,), (2,)), ((0,), (0,)))
    return jax.lax.dot_general(b, tb_half, bc, precision=hi,
                               preferred_element_type=jnp.float32)


def _body(t_ref, xa_ref, xb_ref, o_ref):
    t = t_ref[...]
    s = xa_ref.shape[0]
    tbf = jnp.broadcast_to(t, (s, 32, 64)).astype(jnp.bfloat16)
    c = (_half(xa_ref[:, 0], t, tbf[:, :, :32])
         + _half(xb_ref[:, 0], t, tbf[:, :, 32:]))
    o_ref[...] = c


def kernel(x):
    b, ch = x.shape[0], x.shape[1]
    n = b * ch
    xf = x.reshape(n, 2, 32, 64, 64)
    out = pl.pallas_call(
        _body,
        grid=(n // _S,),
        in_specs=[
            pl.BlockSpec((32, 64), lambda i: (0, 0)),
            pl.BlockSpec((_S, 1, 32, 64, 64), lambda i: (i, 0, 0, 0, 0)),
            pl.BlockSpec((_S, 1, 32, 64, 64), lambda i: (i, 1, 0, 0, 0)),
        ],
        out_specs=pl.BlockSpec((_S, 32, 32, 32), lambda i: (i, 0, 0, 0)),
        out_shape=jax.ShapeDtypeStruct((n, 32, 32, 32), jnp.float32),
        compiler_params=pltpu.CompilerParams(
            dimension_semantics=("parallel",),
        ),
    )(jnp.asarray(_T), xf, xf)
    return jnp.transpose(out, (0, 3, 2, 1)).reshape(b, ch, 32, 32, 32)


# speedup vs baseline: 1.1975x; 1.0856x over previous
import jax
import jax.numpy as jnp
import numpy as np
from jax.experimental import pallas as pl
from jax.experimental.pallas import tpu as pltpu


def _dct_mat(n):
    i = np.arange(n, dtype=np.float64)
    k = np.arange(n, dtype=np.float64)[:, None]
    m = np.cos(np.pi * (2.0 * i[None, :] + 1.0) * k / (2.0 * n))
    s = np.where(k == 0, np.sqrt(1.0 / n), np.sqrt(2.0 / n))
    return m * s


_T = np.asarray(_dct_mat(32)[:28, :].T @ _dct_mat(64)[:28, :], dtype=np.float32)


_S = 8


def _body(t_ref, x_ref, o_ref):
    t = t_ref[...]
    x = x_ref[...]
    s = x.shape[0]
    nt = (((1,), (1,)), ((), ()))
    on = (((1,), (0,)), ((), ()))
    hi = jax.lax.Precision.DEFAULT
    a = jax.lax.dot_general(x.reshape(s * 64 * 64, 64), t, nt, precision=hi,
                            preferred_element_type=jnp.float32)
    a = jnp.swapaxes(a.reshape(s * 64, 64, 32).astype(jnp.bfloat16), 1, 2)
    b = jax.lax.dot_general(a.reshape(s * 64 * 32, 64), t.astype(jnp.bfloat16), nt, precision=hi,
                            preferred_element_type=jnp.float32)
    b = b.reshape(s, 64, 32, 32).astype(jnp.bfloat16)
    bc = (((1,), (2,)), ((0,), (0,)))
    tb = jnp.broadcast_to(t, (s, 32, 64)).astype(jnp.bfloat16)
    c = jax.lax.dot_general(b, tb, bc, precision=hi,
                            preferred_element_type=jnp.float32)
    o_ref[...] = c


def kernel(x):
    b, ch = x.shape[0], x.shape[1]
    n = b * ch
    xf = x.reshape(n, 64, 64, 64)
    out = pl.pallas_call(
        _body,
        grid=(n // _S,),
        in_specs=[
            pl.BlockSpec((32, 64), lambda i: (0, 0)),
            pl.BlockSpec((_S, 64, 64, 64), lambda i: (i, 0, 0, 0)),
        ],
        out_specs=pl.BlockSpec((_S, 32, 32, 32), lambda i: (i, 0, 0, 0)),
        out_shape=jax.ShapeDtypeStruct((n, 32, 32, 32), jnp.float32),
        compiler_params=pltpu.CompilerParams(
            dimension_semantics=("parallel",),
        ),
    )(jnp.asarray(_T), xf)
    return jnp.transpose(out, (0, 3, 2, 1)).reshape(b, ch, 32, 32, 32)
